# baseline (speedup 1.0000x reference)
import functools

import jax
import jax.numpy as jnp
from jax import lax
from jax.experimental import pallas as pl
from jax.experimental.pallas import tpu as pltpu
from jax.experimental.pallas import tpu_sc as plsc

N = 10000
E = 320000
D = 128

NC = 2
NS = 16
NW = NC * NS
CHUNK = E // NW
K = 125
NBLK = CHUNK // K
RPT0 = 624
NTAIL0 = NS * RPT0
NTAIL = N - NTAIL0
DW = 128


def _tiled_copy(src, dst, sid):
    r0 = pl.multiple_of(sid * RPT0, 8)
    pltpu.sync_copy(src.at[pl.ds(r0, RPT0)], dst.at[pl.ds(r0, RPT0)])

    @pl.when(sid == NS - 1)
    def _():
        pltpu.sync_copy(src.at[pl.ds(NTAIL0, NTAIL)], dst.at[pl.ds(NTAIL0, NTAIL)])


@functools.cache
def _deg_kernel():
    mesh = plsc.VectorSubcoreMesh(core_axis_name="c", subcore_axis_name="s")
    return functools.partial(
        pl.kernel,
        mesh=mesh,
        out_type=jax.ShapeDtypeStruct((NC, N, DW), jnp.float32),
        scratch_types=[
            pltpu.VMEM((NBLK, K), jnp.int32),
            pltpu.VMEM((K, DW), jnp.float32),
            pltpu.VMEM_SHARED((N, DW), jnp.float32),
        ],
    )(_deg_body)


def _deg_body(col_hbm, ones_hbm, zeros_hbm, out_hbm, cols_v, ones_v, acc_sh):
    cid = lax.axis_index("c")
    sid = lax.axis_index("s")
    wid = sid * NC + cid
    _tiled_copy(zeros_hbm, acc_sh, sid)
    pltpu.sync_copy(ones_hbm, ones_v)
    pltpu.sync_copy(col_hbm.at[wid], cols_v)
    plsc.subcore_barrier()

    def body(j, carry):
        pltpu.sync_copy(ones_v, acc_sh.at[cols_v.at[j]], add=True)
        return carry

    lax.fori_loop(0, NBLK, body, 0)
    plsc.subcore_barrier()
    _tiled_copy(acc_sh, out_hbm.at[cid], sid)


@functools.cache
def _scatter_kernel():
    mesh = plsc.VectorSubcoreMesh(core_axis_name="c", subcore_axis_name="s")
    return functools.partial(
        pl.kernel,
        mesh=mesh,
        out_type=jax.ShapeDtypeStruct((NC, N, D), jnp.float32),
        scratch_types=[
            pltpu.VMEM((4, K), jnp.int32),
            pltpu.VMEM((4, K), jnp.int32),
            pltpu.VMEM((3, K, D), jnp.float32),
            pltpu.VMEM_SHARED((N, D), jnp.float32),
            pltpu.SemaphoreType.DMA,
            pltpu.SemaphoreType.DMA,
            pltpu.SemaphoreType.DMA,
        ],
    )(_scatter_body)


def _scatter_body(g_hbm, row_hbm, col_hbm, zeros_hbm, out_hbm,
                  rows_v, cols_v, buf_v, acc_sh, gsem, isem, ssem):
    cid = lax.axis_index("c")
    sid = lax.axis_index("s")
    wid = sid * NC + cid

    @pl.when(cid == 0)
    def _():
        _tiled_copy(g_hbm, acc_sh, sid)

    @pl.when(cid == 1)
    def _():
        _tiled_copy(zeros_hbm, acc_sh, sid)

    plsc.subcore_barrier()

    pltpu.sync_copy(row_hbm.at[wid].at[0], rows_v.at[0])
    pltpu.sync_copy(col_hbm.at[wid].at[0], cols_v.at[0])
    pltpu.async_copy(g_hbm.at[rows_v.at[0]], buf_v.at[0], gsem)
    pltpu.async_copy(row_hbm.at[wid].at[1], rows_v.at[1], isem)
    pltpu.async_copy(col_hbm.at[wid].at[1], cols_v.at[1], isem)

    def _drain_scatter():
        pltpu.make_async_copy(g_hbm.at[rows_v.at[0]], buf_v.at[0],
                              ssem).wait()

    def body(j, carry):
        slot = lax.rem(j, 3)
        islot = lax.rem(j, 4)
        pltpu.make_async_copy(g_hbm.at[rows_v.at[islot]], buf_v.at[slot],
                              gsem).wait()

        @pl.when(j >= 2)
        def _():
            _drain_scatter()

        pltpu.async_copy(buf_v.at[slot], acc_sh.at[cols_v.at[islot]], ssem,
                         add=True)

        @pl.when(j + 1 < NBLK)
        def _():
            ns = lax.rem(j + 1, 4)
            pltpu.make_async_copy(row_hbm.at[wid].at[j + 1], rows_v.at[ns],
                                  isem).wait()
            pltpu.make_async_copy(col_hbm.at[wid].at[j + 1], cols_v.at[ns],
                                  isem).wait()
            pltpu.async_copy(g_hbm.at[rows_v.at[ns]], buf_v.at[lax.rem(j + 1, 3)],
                             gsem)

        @pl.when(j + 2 < NBLK)
        def _():
            ps = lax.rem(j + 2, 4)
            pltpu.async_copy(row_hbm.at[wid].at[j + 2], rows_v.at[ps], isem)
            pltpu.async_copy(col_hbm.at[wid].at[j + 2], cols_v.at[ps], isem)

        return carry

    lax.fori_loop(0, NBLK, body, 0)
    _drain_scatter()
    _drain_scatter()
    plsc.subcore_barrier()
    _tiled_copy(acc_sh, out_hbm.at[cid], sid)


_RB = 1000


def _scale_body(dp_ref, x_ref, w_ref, dis_ref, g_ref):
    deg = dp_ref[0] + dp_ref[1] + 1.0
    dis = lax.rsqrt(deg)
    dis_ref[...] = dis
    h = jnp.dot(x_ref[...], w_ref[...], preferred_element_type=jnp.float32)
    g_ref[...] = h * dis[:, 0:1]


def _scale(dp, x, w):
    return pl.pallas_call(
        _scale_body,
        grid=(N // _RB,),
        in_specs=[
            pl.BlockSpec((NC, _RB, DW), lambda i: (0, i, 0)),
            pl.BlockSpec((_RB, D), lambda i: (i, 0)),
            pl.BlockSpec((D, D), lambda i: (0, 0)),
        ],
        out_specs=[
            pl.BlockSpec((_RB, DW), lambda i: (i, 0)),
            pl.BlockSpec((_RB, D), lambda i: (i, 0)),
        ],
        out_shape=[
            jax.ShapeDtypeStruct((N, DW), jnp.float32),
            jax.ShapeDtypeStruct((N, D), jnp.float32),
        ],
    )(dp, x, w)


def _mid_body(p_ref, dis_ref, w_ref, b_ref, g_ref):
    dis = dis_ref[...][:, 0:1]
    o1 = dis * (p_ref[0] + p_ref[1]) + b_ref[...]
    o1 = jnp.maximum(o1, 0.0)
    g_ref[...] = jnp.dot(o1, w_ref[...],
                         preferred_element_type=jnp.float32) * dis


def _mid(p, dis, w, b):
    return pl.pallas_call(
        _mid_body,
        grid=(N // _RB,),
        in_specs=[
            pl.BlockSpec((NC, _RB, D), lambda i: (0, i, 0)),
            pl.BlockSpec((_RB, DW), lambda i: (i, 0)),
            pl.BlockSpec((D, D), lambda i: (0, 0)),
            pl.BlockSpec((1, D), lambda i: (0, 0)),
        ],
        out_specs=pl.BlockSpec((_RB, D), lambda i: (i, 0)),
        out_shape=jax.ShapeDtypeStruct((N, D), jnp.float32),
    )(p, dis, w, b)


def _fin_body(p_ref, dis_ref, b_ref, o_ref):
    dis = dis_ref[...][:, 0:1]
    o_ref[...] = dis * (p_ref[0] + p_ref[1]) + b_ref[...]


def _fin(p, dis, b):
    return pl.pallas_call(
        _fin_body,
        grid=(N // _RB,),
        in_specs=[
            pl.BlockSpec((NC, _RB, D), lambda i: (0, i, 0)),
            pl.BlockSpec((_RB, DW), lambda i: (i, 0)),
            pl.BlockSpec((1, D), lambda i: (0, 0)),
        ],
        out_specs=pl.BlockSpec((_RB, D), lambda i: (i, 0)),
        out_shape=jax.ShapeDtypeStruct((N, D), jnp.float32),
    )(p, dis, b)


@jax.jit
def kernel(x, edge_index, W1, b1, W2, b2):
    row = edge_index[0].reshape(NW, NBLK, K)
    col = edge_index[1].reshape(NW, NBLK, K)
    zeros2d = jnp.zeros((N, D), jnp.float32)
    zeros16 = jnp.zeros((N, DW), jnp.float32)
    ones16 = jnp.ones((K, DW), jnp.float32)

    dp = _deg_kernel()(col, ones16, zeros16)
    dis, g1 = _scale(dp, x, W1)
    p1 = _scatter_kernel()(g1, row, col, zeros2d)
    g2 = _mid(p1, dis, W2, b1.reshape(1, D))
    p2 = _scatter_kernel()(g2, row, col, zeros2d)
    return _fin(p2, dis, b2.reshape(1, D))


# speedup vs baseline: 1.1296x; 1.1296x over previous
import functools

import jax
import jax.numpy as jnp
from jax import lax
from jax.experimental import pallas as pl
from jax.experimental.pallas import tpu as pltpu
from jax.experimental.pallas import tpu_sc as plsc

N = 10000
E = 320000
D = 128

NC = 2
NS = 16
NW = NC * NS
CHUNK = E // NW
K = 125
NBLK = CHUNK // K
RPT0 = 624
NTAIL0 = NS * RPT0
NTAIL = N - NTAIL0
DW = 128


def _tiled_copy(src, dst, sid):
    r0 = pl.multiple_of(sid * RPT0, 8)
    pltpu.sync_copy(src.at[pl.ds(r0, RPT0)], dst.at[pl.ds(r0, RPT0)])

    @pl.when(sid == NS - 1)
    def _():
        pltpu.sync_copy(src.at[pl.ds(NTAIL0, NTAIL)], dst.at[pl.ds(NTAIL0, NTAIL)])


@functools.cache
def _deg_kernel():
    mesh = plsc.VectorSubcoreMesh(core_axis_name="c", subcore_axis_name="s")
    return functools.partial(
        pl.kernel,
        mesh=mesh,
        out_type=jax.ShapeDtypeStruct((NW * N,), jnp.float32),
        scratch_types=[
            pltpu.VMEM((CHUNK,), jnp.int32),
            pltpu.VMEM((N,), jnp.float32),
            pltpu.VMEM((16,), jnp.int32),
        ],
        compiler_params=pltpu.CompilerParams(needs_layout_passes=False),
    )(_deg_body)


def _deg_body(col_hbm, out_hbm, cols_v, hist_v, tmp_v):
    cid = lax.axis_index("c")
    sid = lax.axis_index("s")
    wid = sid * NC + cid
    pltpu.sync_copy(col_hbm.at[wid], cols_v)
    zeros16 = jnp.zeros((16,), jnp.float32)

    def zbody(i, carry):
        hist_v[pl.ds(i * 16, 16)] = zeros16
        return carry

    lax.fori_loop(0, N // 16, zbody, 0)

    iota = lax.iota(jnp.int32, 16)

    def body(i, carry):
        idx = cols_v[pl.ds(i * 16, 16)]
        s = lax.sort(idx)
        tmp_v[...] = s
        prev = plsc.load_gather(tmp_v, [jnp.maximum(iota - 1, 0)])
        nxt = plsc.load_gather(tmp_v, [jnp.minimum(iota + 1, 15)])
        isnew = (s != prev) | (iota == 0)
        islast = (s != nxt) | (iota == 15)
        start = plsc.cummax(jnp.where(isnew, iota, 0))
        cnt = (iota - start + 1).astype(jnp.float32)
        cur = plsc.load_gather(hist_v, [s])
        plsc.store_scatter(hist_v, [s], cur + cnt, mask=islast)
        return carry

    lax.fori_loop(0, CHUNK // 16, body, 0)
    pltpu.sync_copy(hist_v,
                    out_hbm.at[pl.ds(pl.multiple_of(wid * N, 8), N)])


@functools.cache
def _scatter_kernel():
    mesh = plsc.VectorSubcoreMesh(core_axis_name="c", subcore_axis_name="s")
    return functools.partial(
        pl.kernel,
        mesh=mesh,
        out_type=jax.ShapeDtypeStruct((NC, N, D), jnp.float32),
        scratch_types=[
            pltpu.VMEM((4, K), jnp.int32),
            pltpu.VMEM((4, K), jnp.int32),
            pltpu.VMEM((3, K, D), jnp.float32),
            pltpu.VMEM_SHARED((N, D), jnp.float32),
            pltpu.SemaphoreType.DMA,
            pltpu.SemaphoreType.DMA,
            pltpu.SemaphoreType.DMA,
        ],
    )(_scatter_body)


def _scatter_body(g_hbm, row_hbm, col_hbm, zeros_hbm, out_hbm,
                  rows_v, cols_v, buf_v, acc_sh, gsem, isem, ssem):
    cid = lax.axis_index("c")
    sid = lax.axis_index("s")
    wid = sid * NC + cid

    @pl.when(cid == 0)
    def _():
        _tiled_copy(g_hbm, acc_sh, sid)

    @pl.when(cid == 1)
    def _():
        _tiled_copy(zeros_hbm, acc_sh, sid)

    plsc.subcore_barrier()

    pltpu.sync_copy(row_hbm.at[wid].at[0], rows_v.at[0])
    pltpu.sync_copy(col_hbm.at[wid].at[0], cols_v.at[0])
    pltpu.async_copy(g_hbm.at[rows_v.at[0]], buf_v.at[0], gsem)
    pltpu.async_copy(row_hbm.at[wid].at[1], rows_v.at[1], isem)
    pltpu.async_copy(col_hbm.at[wid].at[1], cols_v.at[1], isem)

    def _drain_scatter():
        pltpu.make_async_copy(g_hbm.at[rows_v.at[0]], buf_v.at[0],
                              ssem).wait()

    def body(j, carry):
        slot = lax.rem(j, 3)
        islot = lax.rem(j, 4)
        pltpu.make_async_copy(g_hbm.at[rows_v.at[islot]], buf_v.at[slot],
                              gsem).wait()

        @pl.when(j >= 2)
        def _():
            _drain_scatter()

        pltpu.async_copy(buf_v.at[slot], acc_sh.at[cols_v.at[islot]], ssem,
                         add=True)

        @pl.when(j + 1 < NBLK)
        def _():
            ns = lax.rem(j + 1, 4)
            pltpu.make_async_copy(row_hbm.at[wid].at[j + 1], rows_v.at[ns],
                                  isem).wait()
            pltpu.make_async_copy(col_hbm.at[wid].at[j + 1], cols_v.at[ns],
                                  isem).wait()
            pltpu.async_copy(g_hbm.at[rows_v.at[ns]], buf_v.at[lax.rem(j + 1, 3)],
                             gsem)

        @pl.when(j + 2 < NBLK)
        def _():
            ps = lax.rem(j + 2, 4)
            pltpu.async_copy(row_hbm.at[wid].at[j + 2], rows_v.at[ps], isem)
            pltpu.async_copy(col_hbm.at[wid].at[j + 2], cols_v.at[ps], isem)

        return carry

    lax.fori_loop(0, NBLK, body, 0)
    _drain_scatter()
    _drain_scatter()
    plsc.subcore_barrier()
    _tiled_copy(acc_sh, out_hbm.at[cid], sid)


_RB = 1000


def _scale_body(dp_ref, x_ref, w_ref, dis_ref, g_ref):
    deg = jnp.sum(dp_ref[...], axis=1, keepdims=True) + 1.0
    dis = lax.rsqrt(deg)
    dis_ref[...] = jnp.broadcast_to(dis, (_RB, DW))
    h = jnp.dot(x_ref[...], w_ref[...], preferred_element_type=jnp.float32)
    g_ref[...] = h * dis


def _scale(dp, x, w):
    return pl.pallas_call(
        _scale_body,
        grid=(N // _RB,),
        in_specs=[
            pl.BlockSpec((_RB, NW), lambda i: (i, 0)),
            pl.BlockSpec((_RB, D), lambda i: (i, 0)),
            pl.BlockSpec((D, D), lambda i: (0, 0)),
        ],
        out_specs=[
            pl.BlockSpec((_RB, DW), lambda i: (i, 0)),
            pl.BlockSpec((_RB, D), lambda i: (i, 0)),
        ],
        out_shape=[
            jax.ShapeDtypeStruct((N, DW), jnp.float32),
            jax.ShapeDtypeStruct((N, D), jnp.float32),
        ],
    )(dp, x, w)


def _mid_body(p_ref, dis_ref, w_ref, b_ref, g_ref):
    dis = dis_ref[...][:, 0:1]
    o1 = dis * (p_ref[0] + p_ref[1]) + b_ref[...]
    o1 = jnp.maximum(o1, 0.0)
    g_ref[...] = jnp.dot(o1, w_ref[...],
                         preferred_element_type=jnp.float32) * dis


def _mid(p, dis, w, b):
    return pl.pallas_call(
        _mid_body,
        grid=(N // _RB,),
        in_specs=[
            pl.BlockSpec((NC, _RB, D), lambda i: (0, i, 0)),
            pl.BlockSpec((_RB, DW), lambda i: (i, 0)),
            pl.BlockSpec((D, D), lambda i: (0, 0)),
            pl.BlockSpec((1, D), lambda i: (0, 0)),
        ],
        out_specs=pl.BlockSpec((_RB, D), lambda i: (i, 0)),
        out_shape=jax.ShapeDtypeStruct((N, D), jnp.float32),
    )(p, dis, w, b)


def _fin_body(p_ref, dis_ref, b_ref, o_ref):
    dis = dis_ref[...][:, 0:1]
    o_ref[...] = dis * (p_ref[0] + p_ref[1]) + b_ref[...]


def _fin(p, dis, b):
    return pl.pallas_call(
        _fin_body,
        grid=(N // _RB,),
        in_specs=[
            pl.BlockSpec((NC, _RB, D), lambda i: (0, i, 0)),
            pl.BlockSpec((_RB, DW), lambda i: (i, 0)),
            pl.BlockSpec((1, D), lambda i: (0, 0)),
        ],
        out_specs=pl.BlockSpec((_RB, D), lambda i: (i, 0)),
        out_shape=jax.ShapeDtypeStruct((N, D), jnp.float32),
    )(p, dis, b)


@jax.jit
def kernel(x, edge_index, W1, b1, W2, b2):
    row = edge_index[0].reshape(NW, NBLK, K)
    col = edge_index[1].reshape(NW, NBLK, K)
    col2 = edge_index[1].reshape(NW, CHUNK)
    zeros2d = jnp.zeros((N, D), jnp.float32)

    dp = _deg_kernel()(col2).reshape(NW, N).T
    dis, g1 = _scale(dp, x, W1)
    p1 = _scatter_kernel()(g1, row, col, zeros2d)
    g2 = _mid(p1, dis, W2, b1.reshape(1, D))
    p2 = _scatter_kernel()(g2, row, col, zeros2d)
    return _fin(p2, dis, b2.reshape(1, D))


# speedup vs baseline: 1.1299x; 1.0003x over previous
import functools

import jax
import jax.numpy as jnp
from jax import lax
from jax.experimental import pallas as pl
from jax.experimental.pallas import tpu as pltpu
from jax.experimental.pallas import tpu_sc as plsc

N = 10000
E = 320000
D = 128

NC = 2
NS = 16
NW = NC * NS
CHUNK = E // NW
K = 125
NBLK = CHUNK // K
RPT0 = 624
NTAIL0 = NS * RPT0
NTAIL = N - NTAIL0
DW = 8


def _tiled_copy(src, dst, sid):
    r0 = pl.multiple_of(sid * RPT0, 8)
    pltpu.sync_copy(src.at[pl.ds(r0, RPT0)], dst.at[pl.ds(r0, RPT0)])

    @pl.when(sid == NS - 1)
    def _():
        pltpu.sync_copy(src.at[pl.ds(NTAIL0, NTAIL)], dst.at[pl.ds(NTAIL0, NTAIL)])


@functools.cache
def _deg_kernel():
    mesh = plsc.VectorSubcoreMesh(core_axis_name="c", subcore_axis_name="s")
    return functools.partial(
        pl.kernel,
        mesh=mesh,
        out_type=jax.ShapeDtypeStruct((NW * N,), jnp.float32),
        scratch_types=[
            pltpu.VMEM((CHUNK,), jnp.int32),
            pltpu.VMEM((N,), jnp.float32),
            pltpu.VMEM((16,), jnp.int32),
        ],
        compiler_params=pltpu.CompilerParams(needs_layout_passes=False),
    )(_deg_body)


def _deg_body(col_hbm, out_hbm, cols_v, hist_v, tmp_v):
    cid = lax.axis_index("c")
    sid = lax.axis_index("s")
    wid = sid * NC + cid
    pltpu.sync_copy(col_hbm.at[wid], cols_v)
    zeros16 = jnp.zeros((16,), jnp.float32)

    def zbody(i, carry):
        hist_v[pl.ds(i * 16, 16)] = zeros16
        return carry

    lax.fori_loop(0, N // 16, zbody, 0)

    iota = lax.iota(jnp.int32, 16)

    def body(i, carry):
        idx = cols_v[pl.ds(i * 16, 16)]
        s = lax.sort(idx)
        tmp_v[...] = s
        prev = plsc.load_gather(tmp_v, [jnp.maximum(iota - 1, 0)])
        nxt = plsc.load_gather(tmp_v, [jnp.minimum(iota + 1, 15)])
        isnew = (s != prev) | (iota == 0)
        islast = (s != nxt) | (iota == 15)
        start = plsc.cummax(jnp.where(isnew, iota, 0))
        cnt = (iota - start + 1).astype(jnp.float32)
        cur = plsc.load_gather(hist_v, [s])
        plsc.store_scatter(hist_v, [s], cur + cnt, mask=islast)
        return carry

    lax.fori_loop(0, CHUNK // 16, body, 0)
    pltpu.sync_copy(hist_v,
                    out_hbm.at[pl.ds(pl.multiple_of(wid * N, 8), N)])


@functools.cache
def _scatter_kernel():
    mesh = plsc.VectorSubcoreMesh(core_axis_name="c", subcore_axis_name="s")
    return functools.partial(
        pl.kernel,
        mesh=mesh,
        out_type=jax.ShapeDtypeStruct((NC, N, D), jnp.float32),
        scratch_types=[
            pltpu.VMEM((4, K), jnp.int32),
            pltpu.VMEM((4, K), jnp.int32),
            pltpu.VMEM((3, K, D), jnp.float32),
            pltpu.VMEM_SHARED((N, D), jnp.float32),
            pltpu.SemaphoreType.DMA,
            pltpu.SemaphoreType.DMA,
            pltpu.SemaphoreType.DMA,
        ],
    )(_scatter_body)


def _scatter_body(g_hbm, row_hbm, col_hbm, zeros_hbm, out_hbm,
                  rows_v, cols_v, buf_v, acc_sh, gsem, isem, ssem):
    cid = lax.axis_index("c")
    sid = lax.axis_index("s")
    wid = sid * NC + cid

    @pl.when(cid == 0)
    def _():
        _tiled_copy(g_hbm, acc_sh, sid)

    @pl.when(cid == 1)
    def _():
        _tiled_copy(zeros_hbm, acc_sh, sid)

    plsc.subcore_barrier()

    pltpu.sync_copy(row_hbm.at[wid].at[0], rows_v.at[0])
    pltpu.sync_copy(col_hbm.at[wid].at[0], cols_v.at[0])
    pltpu.async_copy(g_hbm.at[rows_v.at[0]], buf_v.at[0], gsem)
    pltpu.async_copy(row_hbm.at[wid].at[1], rows_v.at[1], isem)
    pltpu.async_copy(col_hbm.at[wid].at[1], cols_v.at[1], isem)

    def _drain_scatter():
        pltpu.make_async_copy(g_hbm.at[rows_v.at[0]], buf_v.at[0],
                              ssem).wait()

    def body(j, carry):
        slot = lax.rem(j, 3)
        islot = lax.rem(j, 4)
        pltpu.make_async_copy(g_hbm.at[rows_v.at[islot]], buf_v.at[slot],
                              gsem).wait()

        @pl.when(j >= 2)
        def _():
            _drain_scatter()

        pltpu.async_copy(buf_v.at[slot], acc_sh.at[cols_v.at[islot]], ssem,
                         add=True)

        @pl.when(j + 1 < NBLK)
        def _():
            ns = lax.rem(j + 1, 4)
            pltpu.make_async_copy(row_hbm.at[wid].at[j + 1], rows_v.at[ns],
                                  isem).wait()
            pltpu.make_async_copy(col_hbm.at[wid].at[j + 1], cols_v.at[ns],
                                  isem).wait()
            pltpu.async_copy(g_hbm.at[rows_v.at[ns]], buf_v.at[lax.rem(j + 1, 3)],
                             gsem)

        @pl.when(j + 2 < NBLK)
        def _():
            ps = lax.rem(j + 2, 4)
            pltpu.async_copy(row_hbm.at[wid].at[j + 2], rows_v.at[ps], isem)
            pltpu.async_copy(col_hbm.at[wid].at[j + 2], cols_v.at[ps], isem)

        return carry

    lax.fori_loop(0, NBLK, body, 0)
    _drain_scatter()
    _drain_scatter()
    plsc.subcore_barrier()
    _tiled_copy(acc_sh, out_hbm.at[cid], sid)


_RB = 1000


def _scale_body(dp_ref, x_ref, w_ref, dis_ref, g_ref):
    deg = jnp.sum(dp_ref[...], axis=1, keepdims=True) + 1.0
    dis = lax.rsqrt(deg)
    dis_ref[...] = jnp.broadcast_to(dis, (_RB, DW))
    h = jnp.dot(x_ref[...], w_ref[...], preferred_element_type=jnp.float32)
    g_ref[...] = h * dis


def _scale(dp, x, w):
    return pl.pallas_call(
        _scale_body,
        grid=(N // _RB,),
        in_specs=[
            pl.BlockSpec((_RB, NW), lambda i: (i, 0)),
            pl.BlockSpec((_RB, D), lambda i: (i, 0)),
            pl.BlockSpec((D, D), lambda i: (0, 0)),
        ],
        out_specs=[
            pl.BlockSpec((_RB, DW), lambda i: (i, 0)),
            pl.BlockSpec((_RB, D), lambda i: (i, 0)),
        ],
        out_shape=[
            jax.ShapeDtypeStruct((N, DW), jnp.float32),
            jax.ShapeDtypeStruct((N, D), jnp.float32),
        ],
    )(dp, x, w)


def _mid_body(p_ref, dis_ref, w_ref, b_ref, g_ref):
    dis = dis_ref[...][:, 0:1]
    o1 = dis * (p_ref[0] + p_ref[1]) + b_ref[...]
    o1 = jnp.maximum(o1, 0.0)
    g_ref[...] = jnp.dot(o1, w_ref[...],
                         preferred_element_type=jnp.float32) * dis


def _mid(p, dis, w, b):
    return pl.pallas_call(
        _mid_body,
        grid=(N // _RB,),
        in_specs=[
            pl.BlockSpec((NC, _RB, D), lambda i: (0, i, 0)),
            pl.BlockSpec((_RB, DW), lambda i: (i, 0)),
            pl.BlockSpec((D, D), lambda i: (0, 0)),
            pl.BlockSpec((1, D), lambda i: (0, 0)),
        ],
        out_specs=pl.BlockSpec((_RB, D), lambda i: (i, 0)),
        out_shape=jax.ShapeDtypeStruct((N, D), jnp.float32),
    )(p, dis, w, b)


def _fin_body(p_ref, dis_ref, b_ref, o_ref):
    dis = dis_ref[...][:, 0:1]
    o_ref[...] = dis * (p_ref[0] + p_ref[1]) + b_ref[...]


def _fin(p, dis, b):
    return pl.pallas_call(
        _fin_body,
        grid=(N // _RB,),
        in_specs=[
            pl.BlockSpec((NC, _RB, D), lambda i: (0, i, 0)),
            pl.BlockSpec((_RB, DW), lambda i: (i, 0)),
            pl.BlockSpec((1, D), lambda i: (0, 0)),
        ],
        out_specs=pl.BlockSpec((_RB, D), lambda i: (i, 0)),
        out_shape=jax.ShapeDtypeStruct((N, D), jnp.float32),
    )(p, dis, b)


@jax.jit
def kernel(x, edge_index, W1, b1, W2, b2):
    row = edge_index[0].reshape(NW, NBLK, K)
    col = edge_index[1].reshape(NW, NBLK, K)
    col2 = edge_index[1].reshape(NW, CHUNK)
    zeros2d = jnp.zeros((N, D), jnp.float32)

    dp = _deg_kernel()(col2).reshape(NW, N).T
    dis, g1 = _scale(dp, x, W1)
    p1 = _scatter_kernel()(g1, row, col, zeros2d)
    g2 = _mid(p1, dis, W2, b1.reshape(1, D))
    p2 = _scatter_kernel()(g2, row, col, zeros2d)
    return _fin(p2, dis, b2.reshape(1, D))


# speedup vs baseline: 1.3506x; 1.1953x over previous
import functools

import jax
import jax.numpy as jnp
from jax import lax
from jax.experimental import pallas as pl
from jax.experimental.pallas import tpu as pltpu
from jax.experimental.pallas import tpu_sc as plsc

N = 10000
E = 320000
D = 128

NC = 2
NS = 16
NW = NC * NS
CHUNK = E // NW
K = 80
NBLK = CHUNK // K
RPT0 = 624
NTAIL0 = NS * RPT0
NTAIL = N - NTAIL0
DW = 8


def _tiled_copy(src, dst, sid):
    r0 = pl.multiple_of(sid * RPT0, 8)
    pltpu.sync_copy(src.at[pl.ds(r0, RPT0)], dst.at[pl.ds(r0, RPT0)])

    @pl.when(sid == NS - 1)
    def _():
        pltpu.sync_copy(src.at[pl.ds(NTAIL0, NTAIL)], dst.at[pl.ds(NTAIL0, NTAIL)])


@functools.cache
def _deg_kernel():
    mesh = plsc.VectorSubcoreMesh(core_axis_name="c", subcore_axis_name="s")
    return functools.partial(
        pl.kernel,
        mesh=mesh,
        out_type=jax.ShapeDtypeStruct((NW * N,), jnp.float32),
        scratch_types=[
            pltpu.VMEM((CHUNK,), jnp.int32),
            pltpu.VMEM((N,), jnp.float32),
            pltpu.VMEM((16,), jnp.int32),
        ],
        compiler_params=pltpu.CompilerParams(needs_layout_passes=False),
    )(_deg_body)


def _deg_body(col_hbm, out_hbm, cols_v, hist_v, tmp_v):
    cid = lax.axis_index("c")
    sid = lax.axis_index("s")
    wid = sid * NC + cid
    pltpu.sync_copy(col_hbm.at[wid], cols_v)
    zeros16 = jnp.zeros((16,), jnp.float32)

    def zbody(i, carry):
        hist_v[pl.ds(i * 16, 16)] = zeros16
        return carry

    lax.fori_loop(0, N // 16, zbody, 0)

    iota = lax.iota(jnp.int32, 16)

    def body(i, carry):
        idx = cols_v[pl.ds(i * 16, 16)]
        s = lax.sort(idx)
        tmp_v[...] = s
        prev = plsc.load_gather(tmp_v, [jnp.maximum(iota - 1, 0)])
        nxt = plsc.load_gather(tmp_v, [jnp.minimum(iota + 1, 15)])
        isnew = (s != prev) | (iota == 0)
        islast = (s != nxt) | (iota == 15)
        start = plsc.cummax(jnp.where(isnew, iota, 0))
        cnt = (iota - start + 1).astype(jnp.float32)
        cur = plsc.load_gather(hist_v, [s])
        plsc.store_scatter(hist_v, [s], cur + cnt, mask=islast)
        return carry

    lax.fori_loop(0, CHUNK // 16, body, 0)
    pltpu.sync_copy(hist_v,
                    out_hbm.at[pl.ds(pl.multiple_of(wid * N, 8), N)])


@functools.cache
def _scatter_kernel():
    mesh = plsc.VectorSubcoreMesh(core_axis_name="c", subcore_axis_name="s")
    return functools.partial(
        pl.kernel,
        mesh=mesh,
        out_type=jax.ShapeDtypeStruct((NC, N, D), jnp.float32),
        scratch_types=[
            pltpu.VMEM((5, K), jnp.int32),
            pltpu.VMEM((5, K), jnp.int32),
            pltpu.VMEM((4, K, D), jnp.float32),
            pltpu.VMEM_SHARED((N, D), jnp.float32),
            pltpu.SemaphoreType.DMA,
            pltpu.SemaphoreType.DMA,
            pltpu.SemaphoreType.DMA,
        ],
    )(_scatter_body)


def _scatter_body(g_hbm, row_hbm, col_hbm, zeros_hbm, out_hbm,
                  rows_v, cols_v, buf_v, acc_sh, gsem, isem, ssem):
    cid = lax.axis_index("c")
    sid = lax.axis_index("s")
    wid = sid * NC + cid

    @pl.when(cid == 0)
    def _():
        _tiled_copy(g_hbm, acc_sh, sid)

    @pl.when(cid == 1)
    def _():
        _tiled_copy(zeros_hbm, acc_sh, sid)

    plsc.subcore_barrier()

    pltpu.sync_copy(row_hbm.at[wid].at[0], rows_v.at[0])
    pltpu.sync_copy(col_hbm.at[wid].at[0], cols_v.at[0])
    pltpu.sync_copy(row_hbm.at[wid].at[1], rows_v.at[1])
    pltpu.sync_copy(col_hbm.at[wid].at[1], cols_v.at[1])
    pltpu.async_copy(g_hbm.at[rows_v.at[0]], buf_v.at[0], gsem)
    pltpu.async_copy(g_hbm.at[rows_v.at[1]], buf_v.at[1], gsem)
    pltpu.async_copy(row_hbm.at[wid].at[2], rows_v.at[2], isem)
    pltpu.async_copy(col_hbm.at[wid].at[2], cols_v.at[2], isem)

    def _drain_scatter():
        pltpu.make_async_copy(g_hbm.at[rows_v.at[0]], buf_v.at[0],
                              ssem).wait()

    def body(j, carry):
        slot = lax.rem(j, 4)
        islot = lax.rem(j, 5)
        pltpu.make_async_copy(g_hbm.at[rows_v.at[islot]], buf_v.at[slot],
                              gsem).wait()

        @pl.when(j >= 2)
        def _():
            _drain_scatter()

        pltpu.async_copy(buf_v.at[slot], acc_sh.at[cols_v.at[islot]], ssem,
                         add=True)

        @pl.when(j + 2 < NBLK)
        def _():
            ns = lax.rem(j + 2, 5)
            pltpu.make_async_copy(row_hbm.at[wid].at[j + 2], rows_v.at[ns],
                                  isem).wait()
            pltpu.make_async_copy(col_hbm.at[wid].at[j + 2], cols_v.at[ns],
                                  isem).wait()
            pltpu.async_copy(g_hbm.at[rows_v.at[ns]], buf_v.at[lax.rem(j + 2, 4)],
                             gsem)

        @pl.when(j + 3 < NBLK)
        def _():
            ps = lax.rem(j + 3, 5)
            pltpu.async_copy(row_hbm.at[wid].at[j + 3], rows_v.at[ps], isem)
            pltpu.async_copy(col_hbm.at[wid].at[j + 3], cols_v.at[ps], isem)

        return carry

    lax.fori_loop(0, NBLK, body, 0)
    _drain_scatter()
    _drain_scatter()
    plsc.subcore_barrier()
    _tiled_copy(acc_sh, out_hbm.at[cid], sid)


_RB = 1000


def _scale_body(dp_ref, x_ref, w_ref, dis_ref, g_ref):
    deg = jnp.sum(dp_ref[...], axis=1, keepdims=True) + 1.0
    dis = lax.rsqrt(deg)
    dis_ref[...] = jnp.broadcast_to(dis, (_RB, DW))
    h = jnp.dot(x_ref[...], w_ref[...], preferred_element_type=jnp.float32)
    g_ref[...] = h * dis


def _scale(dp, x, w):
    return pl.pallas_call(
        _scale_body,
        grid=(N // _RB,),
        in_specs=[
            pl.BlockSpec((_RB, NW), lambda i: (i, 0)),
            pl.BlockSpec((_RB, D), lambda i: (i, 0)),
            pl.BlockSpec((D, D), lambda i: (0, 0)),
        ],
        out_specs=[
            pl.BlockSpec((_RB, DW), lambda i: (i, 0)),
            pl.BlockSpec((_RB, D), lambda i: (i, 0)),
        ],
        out_shape=[
            jax.ShapeDtypeStruct((N, DW), jnp.float32),
            jax.ShapeDtypeStruct((N, D), jnp.float32),
        ],
    )(dp, x, w)


def _mid_body(p_ref, dis_ref, w_ref, b_ref, g_ref):
    dis = dis_ref[...][:, 0:1]
    o1 = dis * (p_ref[0] + p_ref[1]) + b_ref[...]
    o1 = jnp.maximum(o1, 0.0)
    g_ref[...] = jnp.dot(o1, w_ref[...],
                         preferred_element_type=jnp.float32) * dis


def _mid(p, dis, w, b):
    return pl.pallas_call(
        _mid_body,
        grid=(N // _RB,),
        in_specs=[
            pl.BlockSpec((NC, _RB, D), lambda i: (0, i, 0)),
            pl.BlockSpec((_RB, DW), lambda i: (i, 0)),
            pl.BlockSpec((D, D), lambda i: (0, 0)),
            pl.BlockSpec((1, D), lambda i: (0, 0)),
        ],
        out_specs=pl.BlockSpec((_RB, D), lambda i: (i, 0)),
        out_shape=jax.ShapeDtypeStruct((N, D), jnp.float32),
    )(p, dis, w, b)


def _fin_body(p_ref, dis_ref, b_ref, o_ref):
    dis = dis_ref[...][:, 0:1]
    o_ref[...] = dis * (p_ref[0] + p_ref[1]) + b_ref[...]


def _fin(p, dis, b):
    return pl.pallas_call(
        _fin_body,
        grid=(N // _RB,),
        in_specs=[
            pl.BlockSpec((NC, _RB, D), lambda i: (0, i, 0)),
            pl.BlockSpec((_RB, DW), lambda i: (i, 0)),
            pl.BlockSpec((1, D), lambda i: (0, 0)),
        ],
        out_specs=pl.BlockSpec((_RB, D), lambda i: (i, 0)),
        out_shape=jax.ShapeDtypeStruct((N, D), jnp.float32),
    )(p, dis, b)


@jax.jit
def kernel(x, edge_index, W1, b1, W2, b2):
    row = edge_index[0].reshape(NW, NBLK, K)
    col = edge_index[1].reshape(NW, NBLK, K)
    col2 = edge_index[1].reshape(NW, CHUNK)
    zeros2d = jnp.zeros((N, D), jnp.float32)

    dp = _deg_kernel()(col2).reshape(NW, N).T
    dis, g1 = _scale(dp, x, W1)
    p1 = _scatter_kernel()(g1, row, col, zeros2d)
    g2 = _mid(p1, dis, W2, b1.reshape(1, D))
    p2 = _scatter_kernel()(g2, row, col, zeros2d)
    return _fin(p2, dis, b2.reshape(1, D))


# speedup vs baseline: 1.3757x; 1.0186x over previous
import functools

import jax
import jax.numpy as jnp
from jax import lax
from jax.experimental import pallas as pl
from jax.experimental.pallas import tpu as pltpu
from jax.experimental.pallas import tpu_sc as plsc

N = 10000
E = 320000
D = 128

NC = 2
NS = 16
NW = NC * NS
CHUNK = E // NW
K = 80
NBLK = CHUNK // K
RPT0 = 624
NTAIL0 = NS * RPT0
NTAIL = N - NTAIL0
DW = 8


def _tiled_copy(src, dst, sid):
    r0 = pl.multiple_of(sid * RPT0, 8)
    pltpu.sync_copy(src.at[pl.ds(r0, RPT0)], dst.at[pl.ds(r0, RPT0)])

    @pl.when(sid == NS - 1)
    def _():
        pltpu.sync_copy(src.at[pl.ds(NTAIL0, NTAIL)], dst.at[pl.ds(NTAIL0, NTAIL)])


@functools.cache
def _deg_kernel():
    mesh = plsc.VectorSubcoreMesh(core_axis_name="c", subcore_axis_name="s")
    return functools.partial(
        pl.kernel,
        mesh=mesh,
        out_type=jax.ShapeDtypeStruct((NW * N,), jnp.float32),
        scratch_types=[
            pltpu.VMEM((CHUNK,), jnp.int32),
            pltpu.VMEM((N,), jnp.float32),
            pltpu.VMEM((16,), jnp.int32),
        ],
        compiler_params=pltpu.CompilerParams(needs_layout_passes=False),
    )(_deg_body)


def _deg_body(col_hbm, out_hbm, cols_v, hist_v, tmp_v):
    cid = lax.axis_index("c")
    sid = lax.axis_index("s")
    wid = sid * NC + cid
    pltpu.sync_copy(col_hbm.at[wid], cols_v)
    zeros16 = jnp.zeros((16,), jnp.float32)

    def zbody(i, carry):
        hist_v[pl.ds(i * 16, 16)] = zeros16
        return carry

    lax.fori_loop(0, N // 16, zbody, 0)

    iota = lax.iota(jnp.int32, 16)

    def body(i, carry):
        idx = cols_v[pl.ds(i * 16, 16)]
        s = lax.sort(idx)
        tmp_v[...] = s
        prev = plsc.load_gather(tmp_v, [jnp.maximum(iota - 1, 0)])
        nxt = plsc.load_gather(tmp_v, [jnp.minimum(iota + 1, 15)])
        isnew = (s != prev) | (iota == 0)
        islast = (s != nxt) | (iota == 15)
        start = plsc.cummax(jnp.where(isnew, iota, 0))
        cnt = (iota - start + 1).astype(jnp.float32)
        cur = plsc.load_gather(hist_v, [s])
        plsc.store_scatter(hist_v, [s], cur + cnt, mask=islast)
        return carry

    lax.fori_loop(0, CHUNK // 16, body, 0)
    pltpu.sync_copy(hist_v,
                    out_hbm.at[pl.ds(pl.multiple_of(wid * N, 8), N)])


@functools.cache
def _scatter_kernel():
    mesh = plsc.VectorSubcoreMesh(core_axis_name="c", subcore_axis_name="s")
    return functools.partial(
        pl.kernel,
        mesh=mesh,
        out_type=jax.ShapeDtypeStruct((NC, N, D), jnp.float32),
        scratch_types=[
            pltpu.VMEM((5, K), jnp.int32),
            pltpu.VMEM((5, K), jnp.int32),
            pltpu.VMEM((4, K, D), jnp.float32),
            pltpu.VMEM_SHARED((N, D), jnp.float32),
            pltpu.SemaphoreType.DMA,
            pltpu.SemaphoreType.DMA,
            pltpu.SemaphoreType.DMA,
        ],
    )(_scatter_body)


def _scatter_body(g_hbm, row_hbm, col_hbm, zeros_hbm, out_hbm,
                  rows_v, cols_v, buf_v, acc_sh, gsem, isem, ssem):
    cid = lax.axis_index("c")
    sid = lax.axis_index("s")
    wid = sid * NC + cid

    @pl.when(cid == 0)
    def _():
        _tiled_copy(g_hbm, acc_sh, sid)

    @pl.when(cid == 1)
    def _():
        _tiled_copy(zeros_hbm, acc_sh, sid)

    plsc.subcore_barrier()

    pltpu.sync_copy(row_hbm.at[wid].at[0], rows_v.at[0])
    pltpu.sync_copy(col_hbm.at[wid].at[0], cols_v.at[0])
    pltpu.sync_copy(row_hbm.at[wid].at[1], rows_v.at[1])
    pltpu.sync_copy(col_hbm.at[wid].at[1], cols_v.at[1])
    pltpu.sync_copy(row_hbm.at[wid].at[2], rows_v.at[2])
    pltpu.sync_copy(col_hbm.at[wid].at[2], cols_v.at[2])
    pltpu.async_copy(g_hbm.at[rows_v.at[0]], buf_v.at[0], gsem)
    pltpu.async_copy(g_hbm.at[rows_v.at[1]], buf_v.at[1], gsem)
    pltpu.async_copy(g_hbm.at[rows_v.at[2]], buf_v.at[2], gsem)
    pltpu.async_copy(row_hbm.at[wid].at[3], rows_v.at[3], isem)
    pltpu.async_copy(col_hbm.at[wid].at[3], cols_v.at[3], isem)

    def _drain_scatter():
        pltpu.make_async_copy(g_hbm.at[rows_v.at[0]], buf_v.at[0],
                              ssem).wait()

    def body(j, carry):
        slot = lax.rem(j, 4)
        islot = lax.rem(j, 6)
        pltpu.make_async_copy(g_hbm.at[rows_v.at[islot]], buf_v.at[slot],
                              gsem).wait()

        @pl.when(j >= 1)
        def _():
            _drain_scatter()

        pltpu.async_copy(buf_v.at[slot], acc_sh.at[cols_v.at[islot]], ssem,
                         add=True)

        @pl.when(j + 3 < NBLK)
        def _():
            ns = lax.rem(j + 3, 6)
            pltpu.make_async_copy(row_hbm.at[wid].at[j + 3], rows_v.at[ns],
                                  isem).wait()
            pltpu.make_async_copy(col_hbm.at[wid].at[j + 3], cols_v.at[ns],
                                  isem).wait()
            pltpu.async_copy(g_hbm.at[rows_v.at[ns]], buf_v.at[lax.rem(j + 3, 4)],
                             gsem)

        @pl.when(j + 4 < NBLK)
        def _():
            ps = lax.rem(j + 4, 6)
            pltpu.async_copy(row_hbm.at[wid].at[j + 4], rows_v.at[ps], isem)
            pltpu.async_copy(col_hbm.at[wid].at[j + 4], cols_v.at[ps], isem)

        return carry

    lax.fori_loop(0, NBLK, body, 0)
    _drain_scatter()
    plsc.subcore_barrier()
    _tiled_copy(acc_sh, out_hbm.at[cid], sid)


_RB = 1000


def _scale_body(dp_ref, x_ref, w_ref, dis_ref, g_ref):
    deg = jnp.sum(dp_ref[...], axis=1, keepdims=True) + 1.0
    dis = lax.rsqrt(deg)
    dis_ref[...] = jnp.broadcast_to(dis, (_RB, DW))
    h = jnp.dot(x_ref[...], w_ref[...], preferred_element_type=jnp.float32)
    g_ref[...] = h * dis


def _scale(dp, x, w):
    return pl.pallas_call(
        _scale_body,
        grid=(N // _RB,),
        in_specs=[
            pl.BlockSpec((_RB, NW), lambda i: (i, 0)),
            pl.BlockSpec((_RB, D), lambda i: (i, 0)),
            pl.BlockSpec((D, D), lambda i: (0, 0)),
        ],
        out_specs=[
            pl.BlockSpec((_RB, DW), lambda i: (i, 0)),
            pl.BlockSpec((_RB, D), lambda i: (i, 0)),
        ],
        out_shape=[
            jax.ShapeDtypeStruct((N, DW), jnp.float32),
            jax.ShapeDtypeStruct((N, D), jnp.float32),
        ],
    )(dp, x, w)


def _mid_body(p_ref, dis_ref, w_ref, b_ref, g_ref):
    dis = dis_ref[...][:, 0:1]
    o1 = dis * (p_ref[0] + p_ref[1]) + b_ref[...]
    o1 = jnp.maximum(o1, 0.0)
    g_ref[...] = jnp.dot(o1, w_ref[...],
                         preferred_element_type=jnp.float32) * dis


def _mid(p, dis, w, b):
    return pl.pallas_call(
        _mid_body,
        grid=(N // _RB,),
        in_specs=[
            pl.BlockSpec((NC, _RB, D), lambda i: (0, i, 0)),
            pl.BlockSpec((_RB, DW), lambda i: (i, 0)),
            pl.BlockSpec((D, D), lambda i: (0, 0)),
            pl.BlockSpec((1, D), lambda i: (0, 0)),
        ],
        out_specs=pl.BlockSpec((_RB, D), lambda i: (i, 0)),
        out_shape=jax.ShapeDtypeStruct((N, D), jnp.float32),
    )(p, dis, w, b)


def _fin_body(p_ref, dis_ref, b_ref, o_ref):
    dis = dis_ref[...][:, 0:1]
    o_ref[...] = dis * (p_ref[0] + p_ref[1]) + b_ref[...]


def _fin(p, dis, b):
    return pl.pallas_call(
        _fin_body,
        grid=(N // _RB,),
        in_specs=[
            pl.BlockSpec((NC, _RB, D), lambda i: (0, i, 0)),
            pl.BlockSpec((_RB, DW), lambda i: (i, 0)),
            pl.BlockSpec((1, D), lambda i: (0, 0)),
        ],
        out_specs=pl.BlockSpec((_RB, D), lambda i: (i, 0)),
        out_shape=jax.ShapeDtypeStruct((N, D), jnp.float32),
    )(p, dis, b)


@jax.jit
def kernel(x, edge_index, W1, b1, W2, b2):
    row = edge_index[0].reshape(NW, NBLK, K)
    col = edge_index[1].reshape(NW, NBLK, K)
    col2 = edge_index[1].reshape(NW, CHUNK)
    zeros2d = jnp.zeros((N, D), jnp.float32)

    dp = _deg_kernel()(col2).reshape(NW, N).T
    dis, g1 = _scale(dp, x, W1)
    p1 = _scatter_kernel()(g1, row, col, zeros2d)
    g2 = _mid(p1, dis, W2, b1.reshape(1, D))
    p2 = _scatter_kernel()(g2, row, col, zeros2d)
    return _fin(p2, dis, b2.reshape(1, D))
